# Initial kernel scaffold; baseline (speedup 1.0000x reference)
import jax
import jax.numpy as jnp
from jax.experimental import pallas as pl


def kernel(e, a, angle_index, W1a, b1a, W2a, b2a, W1e, b1e, W2e, b2e):
    raise NotImplementedError("write your pallas kernel here")


# speedup vs baseline: 2.0521x; 2.0521x over previous
import functools

import jax
import jax.numpy as jnp
from jax import lax
from jax.experimental import pallas as pl
from jax.experimental.pallas import tpu as pltpu
from jax.experimental.pallas import tpu_sc as plsc

N_E = 100000
N_A = 640000
D_E = 128
D_A = 16
H_A = 64
H_E = 128

NC = 2
NS = 16
NW = NC * NS

CG = 80
PER_W_G = N_A // NW
ROWS_SC = N_A // NS
ACC_ROWS = N_E // NS
ZR = 250


def _selu(x):
    safe = jnp.minimum(x, 0.0)
    return 1.0507009873554805 * jnp.where(
        x > 0, x, 1.6732632423543772 * (jnp.exp(safe) - 1.0))


def _tc_proj(e, wrc):
    B = 2000

    def body(e_ref, w_ref, t_ref):
        t_ref[...] = jnp.dot(e_ref[...], w_ref[...],
                             preferred_element_type=jnp.float32)

    return pl.pallas_call(
        body,
        grid=(N_E // B,),
        in_specs=[
            pl.BlockSpec((B, D_E), lambda i: (i, 0)),
            pl.BlockSpec((D_E, 2 * H_A), lambda i: (0, 0)),
        ],
        out_specs=pl.BlockSpec((B, 2 * H_A), lambda i: (i, 0)),
        out_shape=jax.ShapeDtypeStruct((N_E, 2 * H_A), jnp.float32),
    )(e, wrc)


def _sc_gather(t, row, col):
    mesh = plsc.VectorSubcoreMesh(core_axis_name="c", subcore_axis_name="s")

    @functools.partial(
        pl.kernel,
        mesh=mesh,
        compiler_params=pltpu.CompilerParams(use_tc_tiling_on_sc=False),
        out_type=jax.ShapeDtypeStruct((N_A // 2, 2 * H_A), jnp.float32),
        scratch_types=[
            pltpu.VMEM((CG,), jnp.int32),
            pltpu.VMEM((CG,), jnp.int32),
            pltpu.VMEM((CG, 2 * H_A), jnp.float32),
            pltpu.VMEM((CG, 2 * H_A), jnp.float32),
            pltpu.VMEM((CG // 2, 2 * H_A), jnp.float32),
            pltpu.SemaphoreType.DMA,
            pltpu.SemaphoreType.DMA,
        ],
    )
    def k(t_hbm, row_hbm, col_hbm, out_hbm, ridx, cidx, rbuf, cbuf, gbuf,
          sem0, sem1):
        wid = lax.axis_index("s") * NC + lax.axis_index("c")
        base = wid * PER_W_G

        def chunk(i, carry):
            st = pl.multiple_of(base + i * CG, 8)
            pltpu.sync_copy(row_hbm.at[pl.ds(st, CG)], ridx)
            pltpu.sync_copy(col_hbm.at[pl.ds(st, CG)], cidx)
            cp0 = pltpu.async_copy(t_hbm.at[ridx], rbuf, sem0)
            cp1 = pltpu.async_copy(t_hbm.at[cidx], cbuf, sem1)
            cp0.wait()
            cp1.wait()

            def rowadd(rr, c2):
                for sub in range(2):
                    for j in range(H_A // 16):
                        dst = pl.ds(sub * H_A + j * 16, 16)
                        gbuf[rr, dst] = (rbuf[2 * rr + sub, pl.ds(j * 16, 16)]
                                         + cbuf[2 * rr + sub,
                                                pl.ds(H_A + j * 16, 16)])
                return c2

            lax.fori_loop(0, CG // 2, rowadd, 0)
            pltpu.sync_copy(gbuf,
                            out_hbm.at[pl.ds(pl.multiple_of(st // 2, 8),
                                             CG // 2)])
            return carry

        lax.fori_loop(0, PER_W_G // CG, chunk, 0)

    return k(t, row, col)


def _tc_angle(g2, a2, wa2, b1a2, w2a2, b2a2):
    B = 4000

    def body(g_ref, a_ref, wa_ref, b1_ref, w2_ref, b2_ref, h_ref, an_ref):
        pre = (g_ref[...]
               + jnp.dot(a_ref[...], wa_ref[...],
                         preferred_element_type=jnp.float32)
               + b1_ref[...])
        h = _selu(pre)
        h_ref[...] = h
        an_ref[...] = (jnp.dot(h, w2_ref[...],
                               preferred_element_type=jnp.float32)
                       + b2_ref[...])

    M = N_A // 2
    return pl.pallas_call(
        body,
        grid=(M // B,),
        in_specs=[
            pl.BlockSpec((B, 2 * H_A), lambda i: (i, 0)),
            pl.BlockSpec((B, 2 * D_A), lambda i: (i, 0)),
            pl.BlockSpec((2 * D_A, 2 * H_A), lambda i: (0, 0)),
            pl.BlockSpec((1, 2 * H_A), lambda i: (0, 0)),
            pl.BlockSpec((2 * H_A, 2 * H_A), lambda i: (0, 0)),
            pl.BlockSpec((1, 2 * H_A), lambda i: (0, 0)),
        ],
        out_specs=[
            pl.BlockSpec((B, 2 * H_A), lambda i: (i, 0)),
            pl.BlockSpec((B, 2 * H_A), lambda i: (i, 0)),
        ],
        out_shape=[
            jax.ShapeDtypeStruct((M, 2 * H_A), jnp.float32),
            jax.ShapeDtypeStruct((M, 2 * H_A), jnp.float32),
        ],
    )(g2, a2, wa2, b1a2, w2a2, b2a2)


def _sc_scatter(h2, col):
    mesh = plsc.VectorSubcoreMesh(core_axis_name="c", subcore_axis_name="s")

    @functools.partial(
        pl.kernel,
        mesh=mesh,
        compiler_params=pltpu.CompilerParams(use_tc_tiling_on_sc=False),
        out_type=jax.ShapeDtypeStruct((4 * N_E * 16,), jnp.float32),
        scratch_types=[
            pltpu.VMEM_SHARED((N_E, 16), jnp.float32),
            pltpu.VMEM((CG,), jnp.int32),
            pltpu.VMEM((CG // 2, 2 * H_A), jnp.float32),
            pltpu.VMEM((CG, 16), jnp.float32),
            pltpu.VMEM((ZR, 16), jnp.float32),
            pltpu.VMEM((ZR, 16), jnp.float32),
            pltpu.VMEM((ZR * 16,), jnp.float32),
        ],
    )
    def k(h_hbm, col_hbm, sums_hbm, acc, cidx, hrows, hbuf, zbuf, dbuf, fbuf):
        c = lax.axis_index("c")
        s = lax.axis_index("s")

        def zfill(r, carry):
            zbuf[r, pl.ds(0, 16)] = jnp.zeros((16,), jnp.float32)
            return carry

        lax.fori_loop(0, ZR, zfill, 0)

        def zero_acc():
            for z in range(ACC_ROWS // ZR):
                pltpu.sync_copy(zbuf, acc.at[pl.ds(s * ACC_ROWS + z * ZR, ZR)])

        def accum(grp):
            def chunk(i, carry):
                st = pl.multiple_of(s * ROWS_SC + i * CG, 8)
                pltpu.sync_copy(col_hbm.at[pl.ds(st, CG)], cidx)
                pltpu.sync_copy(
                    h_hbm.at[pl.ds(pl.multiple_of(st // 2, 8), CG // 2)],
                    hrows)

                def rep(rr, c2):
                    for sub in range(2):
                        hbuf[2 * rr + sub, pl.ds(0, 16)] = (
                            hrows[rr, pl.ds(sub * H_A + grp * 16, 16)])
                    return c2

                lax.fori_loop(0, CG // 2, rep, 0)
                pltpu.sync_copy(hbuf, acc.at[cidx], add=True)
                return carry

            lax.fori_loop(0, ROWS_SC // CG, chunk, 0)

        def dump(grp):
            for z in range(ACC_ROWS // ZR):
                r0 = s * ACC_ROWS + z * ZR
                pltpu.sync_copy(acc.at[pl.ds(r0, ZR)], dbuf)

                def rep(r, c2):
                    fbuf[pl.ds(r * 16, 16)] = dbuf[r, pl.ds(0, 16)]
                    return c2

                lax.fori_loop(0, ZR, rep, 0)
                pltpu.sync_copy(
                    fbuf,
                    sums_hbm.at[pl.ds(
                        pl.multiple_of(grp * N_E * 16 + r0 * 16, 8),
                        ZR * 16)])

        for pno in range(2):
            zero_acc()
            plsc.subcore_barrier()
            for cc in range(2):
                @pl.when(c == cc)
                def _(cc=cc):
                    accum(cc * 2 + pno)
            plsc.subcore_barrier()
            for cc in range(2):
                @pl.when(c == cc)
                def _(cc=cc):
                    dump(cc * 2 + pno)
            plsc.subcore_barrier()

    return k(h2, col)


def _sc_counts(col):
    mesh = plsc.VectorSubcoreMesh(core_axis_name="c", subcore_axis_name="s")
    per_w = N_A // NW

    @functools.partial(
        pl.kernel,
        mesh=mesh,
        compiler_params=pltpu.CompilerParams(use_tc_tiling_on_sc=False),
        out_type=jax.ShapeDtypeStruct((NC * N_E * 16,), jnp.float32),
        scratch_types=[
            pltpu.VMEM_SHARED((N_E, 16), jnp.float32),
            pltpu.VMEM((CG,), jnp.int32),
            pltpu.VMEM((CG, 16), jnp.float32),
            pltpu.VMEM((ZR, 16), jnp.float32),
            pltpu.VMEM((ZR, 16), jnp.float32),
            pltpu.VMEM((ZR * 16,), jnp.float32),
        ],
    )
    def k(col_hbm, cnt_hbm, acc, cidx, obuf, zbuf, dbuf, fbuf):
        c = lax.axis_index("c")
        s = lax.axis_index("s")

        def ofill(r, carry):
            obuf[r, pl.ds(0, 16)] = jnp.full((16,), 1.0, jnp.float32)
            return carry

        lax.fori_loop(0, CG, ofill, 0)

        def zfill(r, carry):
            zbuf[r, pl.ds(0, 16)] = jnp.zeros((16,), jnp.float32)
            return carry

        lax.fori_loop(0, ZR, zfill, 0)

        for z in range(ACC_ROWS // ZR):
            pltpu.sync_copy(zbuf, acc.at[pl.ds(s * ACC_ROWS + z * ZR, ZR)])
        plsc.subcore_barrier()

        def chunk(i, carry):
            st = pl.multiple_of(c * (N_A // NC) + s * per_w + i * CG, 8)
            pltpu.sync_copy(col_hbm.at[pl.ds(st, CG)], cidx)
            pltpu.sync_copy(obuf, acc.at[cidx], add=True)
            return carry

        lax.fori_loop(0, per_w // CG, chunk, 0)
        plsc.subcore_barrier()

        for cc in range(2):
            @pl.when(c == cc)
            def _(cc=cc):
                for z in range(ACC_ROWS // ZR):
                    r0 = s * ACC_ROWS + z * ZR
                    pltpu.sync_copy(acc.at[pl.ds(r0, ZR)], dbuf)

                    def rep(r, c2):
                        fbuf[pl.ds(r * 16, 16)] = dbuf[r, pl.ds(0, 16)]
                        return c2

                    lax.fori_loop(0, ZR, rep, 0)
                    pltpu.sync_copy(
                        fbuf,
                        cnt_hbm.at[pl.ds(
                            pl.multiple_of(cc * N_E * 16 + r0 * 16, 8),
                            ZR * 16)])

    return k(col)


def _tc_edge(sums, cnt64, e, w2a, b2a, wea, wee, b1e, w2e, b2e):
    B = 2000

    def body(s_ref, c_ref, e_ref, w2a_ref, b2a_ref, wea_ref, wee_ref,
             b1e_ref, w2e_ref, b2e_ref, out_ref):
        cnt = c_ref[...]
        mean = s_ref[...] / jnp.maximum(cnt, 1.0)
        aggr = (jnp.dot(mean, w2a_ref[...], preferred_element_type=jnp.float32)
                + b2a_ref[...])
        aggr = jnp.where(cnt > 0, aggr, 0.0)
        he = _selu(
            jnp.dot(aggr, wea_ref[...], preferred_element_type=jnp.float32)
            + jnp.dot(e_ref[...], wee_ref[...],
                      preferred_element_type=jnp.float32)
            + b1e_ref[...])
        out_ref[...] = (jnp.dot(he, w2e_ref[...],
                                preferred_element_type=jnp.float32)
                        + b2e_ref[...])

    return pl.pallas_call(
        body,
        grid=(N_E // B,),
        in_specs=[
            pl.BlockSpec((B, H_A), lambda i: (i, 0)),
            pl.BlockSpec((B, H_A), lambda i: (i, 0)),
            pl.BlockSpec((B, D_E), lambda i: (i, 0)),
            pl.BlockSpec((H_A, H_A), lambda i: (0, 0)),
            pl.BlockSpec((1, H_A), lambda i: (0, 0)),
            pl.BlockSpec((H_A, H_E), lambda i: (0, 0)),
            pl.BlockSpec((D_E, H_E), lambda i: (0, 0)),
            pl.BlockSpec((1, H_E), lambda i: (0, 0)),
            pl.BlockSpec((H_E, H_E), lambda i: (0, 0)),
            pl.BlockSpec((1, H_E), lambda i: (0, 0)),
        ],
        out_specs=pl.BlockSpec((B, H_E), lambda i: (i, 0)),
        out_shape=jax.ShapeDtypeStruct((N_E, H_E), jnp.float32),
    )(sums, cnt64, e, w2a, b2a, wea, wee, b1e, w2e, b2e)


def _blockdiag2(w):
    fi, fo = w.shape
    z = jnp.zeros((fi, fo), w.dtype)
    return jnp.concatenate([
        jnp.concatenate([w, z], axis=1),
        jnp.concatenate([z, w], axis=1),
    ], axis=0)


def kernel(e, a, angle_index, W1a, b1a, W2a, b2a, W1e, b1e, W2e, b2e):
    row = angle_index[0]
    col = angle_index[1]
    wa = W1a[:D_A]
    wrc = jnp.concatenate(
        [W1a[D_A:D_A + D_E], W1a[D_A + D_E:]], axis=1)

    t = _tc_proj(e, wrc)
    g2 = _sc_gather(t, row, col)
    a2 = a.reshape(N_A // 2, 2 * D_A)
    wa2 = _blockdiag2(wa)
    w2a2 = _blockdiag2(W2a)
    b1a2 = jnp.tile(b1a, 2).reshape(1, 2 * H_A)
    b2a2 = jnp.tile(b2a, 2).reshape(1, 2 * H_A)
    h2, an2 = _tc_angle(g2, a2, wa2, b1a2, w2a2, b2a2)
    a_new = an2.reshape(N_A, H_A)

    cnts = _sc_counts(col)
    sums4 = _sc_scatter(h2, col)

    s4 = sums4.reshape(4, N_E, 16)
    sums = jnp.concatenate([s4[0], s4[1], s4[2], s4[3]], axis=1)
    c2 = cnts.reshape(2, N_E, 16)
    cnt1 = c2[0, :, 0] + c2[1, :, 0]
    cnt64 = jnp.broadcast_to(cnt1[:, None], (N_E, H_A))

    e_new = _tc_edge(sums, cnt64, e, W2a, b2a.reshape(1, H_A),
                     W1e[:H_A], W1e[H_A:], b1e.reshape(1, H_E), W2e,
                     b2e.reshape(1, H_E))
    return (e_new, a_new)


# speedup vs baseline: 3.0296x; 1.4764x over previous
import functools

import jax
import jax.numpy as jnp
from jax import lax
from jax.experimental import pallas as pl
from jax.experimental.pallas import tpu as pltpu
from jax.experimental.pallas import tpu_sc as plsc

N_E = 100000
N_A = 640000
D_E = 128
D_A = 16
H_A = 64
H_E = 128

NC = 2
NS = 16
NW = NC * NS

CG = 80
PER_W_G = N_A // NW
ROWS_SC = N_A // NS
ACC_ROWS = N_E // NS
ZR = 250


def _selu(x):
    safe = jnp.minimum(x, 0.0)
    return 1.0507009873554805 * jnp.where(
        x > 0, x, 1.6732632423543772 * (jnp.exp(safe) - 1.0))


def _tc_proj(e, wrc):
    B = 2000

    def body(e_ref, w_ref, t_ref):
        t_ref[...] = jnp.dot(e_ref[...], w_ref[...],
                             preferred_element_type=jnp.float32)

    return pl.pallas_call(
        body,
        grid=(N_E // B,),
        in_specs=[
            pl.BlockSpec((B, D_E), lambda i: (i, 0)),
            pl.BlockSpec((D_E, 2 * H_A), lambda i: (0, 0)),
        ],
        out_specs=pl.BlockSpec((B, 2 * H_A), lambda i: (i, 0)),
        out_shape=jax.ShapeDtypeStruct((N_E, 2 * H_A), jnp.float32),
    )(e, wrc)


def _sc_gather(t, row, col):
    mesh = plsc.VectorSubcoreMesh(core_axis_name="c", subcore_axis_name="s")

    @functools.partial(
        pl.kernel,
        mesh=mesh,
        compiler_params=pltpu.CompilerParams(use_tc_tiling_on_sc=False),
        out_type=jax.ShapeDtypeStruct((N_A // 2, 2 * H_A), jnp.float32),
        scratch_types=[
            pltpu.VMEM((2, CG), jnp.int32),
            pltpu.VMEM((2, CG), jnp.int32),
            pltpu.VMEM((2, CG, 2 * H_A), jnp.float32),
            pltpu.VMEM((2, CG, 2 * H_A), jnp.float32),
            pltpu.VMEM((CG // 2, 2 * H_A), jnp.float32),
            pltpu.SemaphoreType.DMA,
            pltpu.SemaphoreType.DMA,
            pltpu.SemaphoreType.DMA,
            pltpu.SemaphoreType.DMA,
        ],
    )
    def k(t_hbm, row_hbm, col_hbm, out_hbm, ridx, cidx, rbuf, cbuf, gbuf,
          semi0, semi1, semg0, semg1):
        wid = lax.axis_index("s") * NC + lax.axis_index("c")
        base = wid * PER_W_G
        NCH = PER_W_G // CG
        semi = (semi0, semi1)
        semg = (semg0, semg1)

        def issue_idx(i, b):
            st = pl.multiple_of(base + i * CG, 8)
            pltpu.async_copy(row_hbm.at[pl.ds(st, CG)], ridx.at[b], semi[b])
            pltpu.async_copy(col_hbm.at[pl.ds(st, CG)], cidx.at[b], semi[b])

        def wait_idx(i, b):
            st = pl.multiple_of(base + i * CG, 8)
            pltpu.make_async_copy(row_hbm.at[pl.ds(st, CG)], ridx.at[b],
                                  semi[b]).wait()
            pltpu.make_async_copy(col_hbm.at[pl.ds(st, CG)], cidx.at[b],
                                  semi[b]).wait()

        def issue_gather(b):
            pltpu.async_copy(t_hbm.at[ridx.at[b]], rbuf.at[b], semg[b])
            pltpu.async_copy(t_hbm.at[cidx.at[b]], cbuf.at[b], semg[b])

        def wait_gather(b):
            pltpu.make_async_copy(t_hbm.at[ridx.at[b]], rbuf.at[b],
                                  semg[b]).wait()
            pltpu.make_async_copy(t_hbm.at[cidx.at[b]], cbuf.at[b],
                                  semg[b]).wait()

        issue_idx(0, 0)
        wait_idx(0, 0)
        issue_gather(0)
        issue_idx(1, 1)

        def chunk(i, carry):
            b = lax.rem(i, 2)

            def proc(b):
                wait_gather(b)

                @pl.when(i + 1 < NCH)
                def _():
                    wait_idx(i + 1, 1 - b)
                    issue_gather(1 - b)

                @pl.when(i + 2 < NCH)
                def _():
                    issue_idx(i + 2, b)

                def rowadd(rr, c2):
                    for sub in range(2):
                        for j in range(H_A // 16):
                            dst = pl.ds(sub * H_A + j * 16, 16)
                            gbuf[rr, dst] = (
                                rbuf[b, 2 * rr + sub, pl.ds(j * 16, 16)]
                                + cbuf[b, 2 * rr + sub,
                                       pl.ds(H_A + j * 16, 16)])
                    return c2

                lax.fori_loop(0, CG // 2, rowadd, 0)
                st = pl.multiple_of(base + i * CG, 8)
                pltpu.sync_copy(gbuf,
                                out_hbm.at[pl.ds(pl.multiple_of(st // 2, 8),
                                                 CG // 2)])

            for bb in range(2):
                @pl.when(b == bb)
                def _(bb=bb):
                    proc(bb)
            return carry

        lax.fori_loop(0, NCH, chunk, 0)

    return k(t, row, col)


def _tc_angle(g2, a2, wa2, b1a2, w2a2, b2a2):
    B = 2000

    def body(g_ref, a_ref, wa_ref, b1_ref, w2_ref, b2_ref, an_ref,
             hg0_ref, hg1_ref, hg2_ref, hg3_ref):
        pre = (g_ref[...]
               + jnp.dot(a_ref[...], wa_ref[...],
                         preferred_element_type=jnp.float32)
               + b1_ref[...])
        h = _selu(pre)
        an_ref[...] = (jnp.dot(h, w2_ref[...],
                               preferred_element_type=jnp.float32)
                       + b2_ref[...])
        for grp, ref in enumerate((hg0_ref, hg1_ref, hg2_ref, hg3_ref)):
            ref[...] = jnp.concatenate(
                [h[:, grp * 16:grp * 16 + 16],
                 h[:, H_A + grp * 16:H_A + grp * 16 + 16]], axis=1)

    M = N_A // 2
    HG = jax.ShapeDtypeStruct((N_A // 2, 32), jnp.float32)
    return pl.pallas_call(
        body,
        grid=(M // B,),
        in_specs=[
            pl.BlockSpec((B, 2 * H_A), lambda i: (i, 0)),
            pl.BlockSpec((B, 2 * D_A), lambda i: (i, 0)),
            pl.BlockSpec((2 * D_A, 2 * H_A), lambda i: (0, 0)),
            pl.BlockSpec((1, 2 * H_A), lambda i: (0, 0)),
            pl.BlockSpec((2 * H_A, 2 * H_A), lambda i: (0, 0)),
            pl.BlockSpec((1, 2 * H_A), lambda i: (0, 0)),
        ],
        out_specs=[
            pl.BlockSpec((B, 2 * H_A), lambda i: (i, 0)),
            pl.BlockSpec((B, 32), lambda i: (i, 0)),
            pl.BlockSpec((B, 32), lambda i: (i, 0)),
            pl.BlockSpec((B, 32), lambda i: (i, 0)),
            pl.BlockSpec((B, 32), lambda i: (i, 0)),
        ],
        out_shape=[
            jax.ShapeDtypeStruct((M, 2 * H_A), jnp.float32),
            HG, HG, HG, HG,
        ],
    )(g2, a2, wa2, b1a2, w2a2, b2a2)


def _sc_scatter(hg0, hg1, hg2, hg3, col):
    mesh = plsc.VectorSubcoreMesh(core_axis_name="c", subcore_axis_name="s")
    NCH = ROWS_SC // CG
    HR = CG // 2

    @functools.partial(
        pl.kernel,
        mesh=mesh,
        compiler_params=pltpu.CompilerParams(use_tc_tiling_on_sc=False),
        out_type=jax.ShapeDtypeStruct((4 * N_E * 16,), jnp.float32),
        scratch_types=[
            pltpu.VMEM_SHARED((N_E, 16), jnp.float32),
            pltpu.VMEM((2, CG), jnp.int32),
            pltpu.VMEM((2, HR, 32), jnp.float32),
            pltpu.VMEM((CG, 16), jnp.float32),
            pltpu.VMEM((ZR, 16), jnp.float32),
            pltpu.VMEM((ZR, 16), jnp.float32),
            pltpu.VMEM((ZR * 16,), jnp.float32),
            pltpu.SemaphoreType.DMA,
            pltpu.SemaphoreType.DMA,
            pltpu.SemaphoreType.DMA,
            pltpu.SemaphoreType.DMA,
        ],
    )
    def k(h0_hbm, h1_hbm, h2_hbm, h3_hbm, col_hbm, sums_hbm, acc, cidx,
          hrows, hbuf, zbuf, dbuf, fbuf, semc0, semc1, semh0, semh1):
        c = lax.axis_index("c")
        s = lax.axis_index("s")
        semc = (semc0, semc1)
        semh = (semh0, semh1)

        def zfill(r, carry):
            zbuf[r, pl.ds(0, 16)] = jnp.zeros((16,), jnp.float32)
            return carry

        lax.fori_loop(0, ZR, zfill, 0)

        def zero_acc():
            for z in range(ACC_ROWS // ZR):
                pltpu.sync_copy(zbuf, acc.at[pl.ds(s * ACC_ROWS + z * ZR, ZR)])

        def accum(grp, h_hbm):
            def issue(i, b):
                st = pl.multiple_of(s * ROWS_SC + i * CG, 8)
                pltpu.async_copy(col_hbm.at[pl.ds(st, CG)], cidx.at[b],
                                 semc[b])
                pltpu.async_copy(
                    h_hbm.at[pl.ds(pl.multiple_of(st // 2, 4), HR)],
                    hrows.at[b], semh[b])

            def wait(i, b):
                st = pl.multiple_of(s * ROWS_SC + i * CG, 8)
                pltpu.make_async_copy(col_hbm.at[pl.ds(st, CG)], cidx.at[b],
                                      semc[b]).wait()
                pltpu.make_async_copy(
                    h_hbm.at[pl.ds(pl.multiple_of(st // 2, 4), HR)],
                    hrows.at[b], semh[b]).wait()

            issue(0, 0)
            issue(1, 1)

            def outer(i2, carry):
                for b in range(2):
                    i = i2 * 2 + b
                    wait(i, b)

                    def rep(rr, c2):
                        for u in range(2):
                            hbuf[2 * rr + u, pl.ds(0, 16)] = (
                                hrows[b, rr, pl.ds(u * 16, 16)])
                        return c2

                    lax.fori_loop(0, HR, rep, 0)
                    pltpu.sync_copy(hbuf, acc.at[cidx.at[b]], add=True)

                    @pl.when(i + 2 < NCH)
                    def _(i=i, b=b):
                        issue(i + 2, b)
                return carry

            lax.fori_loop(0, NCH // 2, outer, 0)

        def dump(grp):
            for z in range(ACC_ROWS // ZR):
                r0 = s * ACC_ROWS + z * ZR
                pltpu.sync_copy(acc.at[pl.ds(r0, ZR)], dbuf)

                def rep(r, c2):
                    fbuf[pl.ds(r * 16, 16)] = dbuf[r, pl.ds(0, 16)]
                    return c2

                lax.fori_loop(0, ZR, rep, 0)
                pltpu.sync_copy(
                    fbuf,
                    sums_hbm.at[pl.ds(
                        pl.multiple_of(grp * N_E * 16 + r0 * 16, 8),
                        ZR * 16)])

        for pno in range(2):
            zero_acc()
            plsc.subcore_barrier()
            h_refs = (h0_hbm, h1_hbm, h2_hbm, h3_hbm)
            for cc in range(2):
                @pl.when(c == cc)
                def _(cc=cc, pno=pno):
                    accum(cc * 2 + pno, h_refs[cc * 2 + pno])
            plsc.subcore_barrier()
            for cc in range(2):
                @pl.when(c == cc)
                def _(cc=cc):
                    dump(cc * 2 + pno)
            plsc.subcore_barrier()

    return k(hg0, hg1, hg2, hg3, col)


def _sc_counts(col):
    mesh = plsc.VectorSubcoreMesh(core_axis_name="c", subcore_axis_name="s")
    per_w = N_A // NW

    @functools.partial(
        pl.kernel,
        mesh=mesh,
        compiler_params=pltpu.CompilerParams(use_tc_tiling_on_sc=False),
        out_type=jax.ShapeDtypeStruct((NC * N_E * 16,), jnp.float32),
        scratch_types=[
            pltpu.VMEM_SHARED((N_E, 16), jnp.float32),
            pltpu.VMEM((CG,), jnp.int32),
            pltpu.VMEM((CG, 16), jnp.float32),
            pltpu.VMEM((ZR, 16), jnp.float32),
            pltpu.VMEM((ZR, 16), jnp.float32),
            pltpu.VMEM((ZR * 16,), jnp.float32),
        ],
    )
    def k(col_hbm, cnt_hbm, acc, cidx, obuf, zbuf, dbuf, fbuf):
        c = lax.axis_index("c")
        s = lax.axis_index("s")

        def ofill(r, carry):
            obuf[r, pl.ds(0, 16)] = jnp.full((16,), 1.0, jnp.float32)
            return carry

        lax.fori_loop(0, CG, ofill, 0)

        def zfill(r, carry):
            zbuf[r, pl.ds(0, 16)] = jnp.zeros((16,), jnp.float32)
            return carry

        lax.fori_loop(0, ZR, zfill, 0)

        for z in range(ACC_ROWS // ZR):
            pltpu.sync_copy(zbuf, acc.at[pl.ds(s * ACC_ROWS + z * ZR, ZR)])
        plsc.subcore_barrier()

        def chunk(i, carry):
            st = pl.multiple_of(c * (N_A // NC) + s * per_w + i * CG, 8)
            pltpu.sync_copy(col_hbm.at[pl.ds(st, CG)], cidx)
            pltpu.sync_copy(obuf, acc.at[cidx], add=True)
            return carry

        lax.fori_loop(0, per_w // CG, chunk, 0)
        plsc.subcore_barrier()

        for cc in range(2):
            @pl.when(c == cc)
            def _(cc=cc):
                for z in range(ACC_ROWS // ZR):
                    r0 = s * ACC_ROWS + z * ZR
                    pltpu.sync_copy(acc.at[pl.ds(r0, ZR)], dbuf)

                    def rep(r, c2):
                        fbuf[pl.ds(r * 16, 16)] = dbuf[r, pl.ds(0, 16)]
                        return c2

                    lax.fori_loop(0, ZR, rep, 0)
                    pltpu.sync_copy(
                        fbuf,
                        cnt_hbm.at[pl.ds(
                            pl.multiple_of(cc * N_E * 16 + r0 * 16, 8),
                            ZR * 16)])

    return k(col)


def _tc_edge(sums, cnt64, e, w2a, b2a, wea, wee, b1e, w2e, b2e):
    B = 2000

    def body(s_ref, c_ref, e_ref, w2a_ref, b2a_ref, wea_ref, wee_ref,
             b1e_ref, w2e_ref, b2e_ref, out_ref):
        cnt = c_ref[...]
        mean = s_ref[...] / jnp.maximum(cnt, 1.0)
        aggr = (jnp.dot(mean, w2a_ref[...], preferred_element_type=jnp.float32)
                + b2a_ref[...])
        aggr = jnp.where(cnt > 0, aggr, 0.0)
        he = _selu(
            jnp.dot(aggr, wea_ref[...], preferred_element_type=jnp.float32)
            + jnp.dot(e_ref[...], wee_ref[...],
                      preferred_element_type=jnp.float32)
            + b1e_ref[...])
        out_ref[...] = (jnp.dot(he, w2e_ref[...],
                                preferred_element_type=jnp.float32)
                        + b2e_ref[...])

    return pl.pallas_call(
        body,
        grid=(N_E // B,),
        in_specs=[
            pl.BlockSpec((B, H_A), lambda i: (i, 0)),
            pl.BlockSpec((B, H_A), lambda i: (i, 0)),
            pl.BlockSpec((B, D_E), lambda i: (i, 0)),
            pl.BlockSpec((H_A, H_A), lambda i: (0, 0)),
            pl.BlockSpec((1, H_A), lambda i: (0, 0)),
            pl.BlockSpec((H_A, H_E), lambda i: (0, 0)),
            pl.BlockSpec((D_E, H_E), lambda i: (0, 0)),
            pl.BlockSpec((1, H_E), lambda i: (0, 0)),
            pl.BlockSpec((H_E, H_E), lambda i: (0, 0)),
            pl.BlockSpec((1, H_E), lambda i: (0, 0)),
        ],
        out_specs=pl.BlockSpec((B, H_E), lambda i: (i, 0)),
        out_shape=jax.ShapeDtypeStruct((N_E, H_E), jnp.float32),
    )(sums, cnt64, e, w2a, b2a, wea, wee, b1e, w2e, b2e)


def _blockdiag2(w):
    fi, fo = w.shape
    z = jnp.zeros((fi, fo), w.dtype)
    return jnp.concatenate([
        jnp.concatenate([w, z], axis=1),
        jnp.concatenate([z, w], axis=1),
    ], axis=0)


def kernel(e, a, angle_index, W1a, b1a, W2a, b2a, W1e, b1e, W2e, b2e):
    row = angle_index[0]
    col = angle_index[1]
    wa = W1a[:D_A]
    wrc = jnp.concatenate(
        [W1a[D_A:D_A + D_E], W1a[D_A + D_E:]], axis=1)

    t = _tc_proj(e, wrc)
    g2 = _sc_gather(t, row, col)
    a2 = a.reshape(N_A // 2, 2 * D_A)
    wa2 = _blockdiag2(wa)
    w2a2 = _blockdiag2(W2a)
    b1a2 = jnp.tile(b1a, 2).reshape(1, 2 * H_A)
    b2a2 = jnp.tile(b2a, 2).reshape(1, 2 * H_A)
    an2, hg0, hg1, hg2, hg3 = _tc_angle(g2, a2, wa2, b1a2, w2a2, b2a2)
    a_new = an2.reshape(N_A, H_A)

    cnts = _sc_counts(col)
    sums4 = _sc_scatter(hg0, hg1, hg2, hg3, col)

    s4 = sums4.reshape(4, N_E, 16)
    sums = jnp.concatenate([s4[0], s4[1], s4[2], s4[3]], axis=1)
    c2 = cnts.reshape(2, N_E, 16)
    cnt1 = c2[0, :, 0] + c2[1, :, 0]
    cnt64 = jnp.broadcast_to(cnt1[:, None], (N_E, H_A))

    e_new = _tc_edge(sums, cnt64, e, W2a, b2a.reshape(1, H_A),
                     W1e[:H_A], W1e[H_A:], b1e.reshape(1, H_E), W2e,
                     b2e.reshape(1, H_E))
    return (e_new, a_new)
